# Initial kernel scaffold; baseline (speedup 1.0000x reference)
import functools

import jax
import jax.numpy as jnp
from jax import lax
from jax.experimental import pallas as pl
from jax.experimental.pallas import tpu as pltpu
from jax.experimental.pallas import tpu_sc as plsc

N_NODES = 50000
N_EDGES = 800000
C = 16

CH = 128
NW = 32
EPW = N_EDGES // NW
G_FULL = EPW // CH
G_TAIL = EPW - G_FULL * CH
NCHUNK = N_EDGES // CH
ROWS_PT = N_NODES // 16
Z_FULL = ROWS_PT // CH
Z_TAIL = ROWS_PT - Z_FULL * CH

_mesh = plsc.VectorSubcoreMesh(core_axis_name="c", subcore_axis_name="s")


@functools.partial(
    pl.kernel,
    mesh=_mesh,
    out_type=jax.ShapeDtypeStruct((N_EDGES, 64), jnp.float32),
    scratch_types=[
        pltpu.VMEM((EPW,), jnp.int32),
        pltpu.VMEM((2, CH, 64), jnp.float32),
        pltpu.VMEM((G_TAIL, 64), jnp.float32),
        pltpu.SemaphoreType.DMA,
        pltpu.SemaphoreType.DMA,
    ],
)
def _gather_kernel(nodes_hbm, idx_hbm, out_hbm, idx_v, rows_v, tail_v, sem_g, sem_w):
    cid = lax.axis_index("c")
    sid = lax.axis_index("s")
    wid = sid * 2 + cid
    base = wid * EPW
    pltpu.sync_copy(idx_hbm.at[pl.ds(base, EPW)], idx_v)

    def pair(g, carry):
        for b in range(2):
            j = g * 2 + b

            @pl.when(j < G_FULL)
            def _(b=b, j=j):
                @pl.when(j >= 2)
                def _():
                    pltpu.make_async_copy(
                        nodes_hbm.at[pl.ds(0, CH)], rows_v.at[b], sem_w
                    ).wait()

                pltpu.async_copy(
                    nodes_hbm.at[idx_v.at[pl.ds(j * CH, CH)]], rows_v.at[b], sem_g
                ).wait()
                pltpu.async_copy(
                    rows_v.at[b], out_hbm.at[pl.ds(base + j * CH, CH)], sem_w
                )

        return carry

    lax.fori_loop(0, (G_FULL + 1) // 2, pair, 0)
    pltpu.make_async_copy(nodes_hbm.at[pl.ds(0, CH)], rows_v.at[0], sem_w).wait()
    pltpu.make_async_copy(nodes_hbm.at[pl.ds(0, CH)], rows_v.at[1], sem_w).wait()
    pltpu.async_copy(
        nodes_hbm.at[idx_v.at[pl.ds(G_FULL * CH, G_TAIL)]], tail_v, sem_g
    ).wait()
    pltpu.sync_copy(tail_v, out_hbm.at[pl.ds(base + G_FULL * CH, G_TAIL)])


@functools.partial(
    pl.kernel,
    mesh=_mesh,
    out_type=jax.ShapeDtypeStruct((2, N_NODES, 32), jnp.float32),
    scratch_types=[
        pltpu.VMEM((2, CH), jnp.int32),
        pltpu.VMEM((2, CH, 32), jnp.float32),
        pltpu.VMEM((CH, 32), jnp.float32),
        pltpu.VMEM_SHARED((N_NODES, 32), jnp.float32),
        pltpu.SemaphoreType.DMA,
    ],
)
def _scatter_kernel(m_hbm, ridx_hbm, zero_hbm, out_hbm, idx_v, msg_v, zero_v, acc, sem_l):
    cid = lax.axis_index("c")
    sid = lax.axis_index("s")
    row0 = sid * ROWS_PT

    pltpu.sync_copy(zero_hbm, zero_v)

    def zbody(j, carry):
        pltpu.sync_copy(zero_v, acc.at[pl.ds(row0 + j * CH, CH)])
        return carry

    lax.fori_loop(0, Z_FULL, zbody, 0)
    pltpu.sync_copy(
        zero_v.at[pl.ds(0, Z_TAIL)], acc.at[pl.ds(row0 + Z_FULL * CH, Z_TAIL)]
    )
    plsc.subcore_barrier()

    pltpu.async_copy(ridx_hbm.at[pl.ds(sid * CH, CH)], idx_v.at[0], sem_l)
    pltpu.async_copy(m_hbm.at[cid, pl.ds(sid * CH, CH)], msg_v.at[0], sem_l)

    def spair(g, carry):
        for b in range(2):
            k = g * 2 + b
            chunk = k * 16 + sid

            @pl.when(chunk < NCHUNK)
            def _(b=b, chunk=chunk):
                pltpu.make_async_copy(
                    ridx_hbm.at[pl.ds(0, CH)], idx_v.at[b], sem_l
                ).wait()
                pltpu.make_async_copy(
                    m_hbm.at[0, pl.ds(0, CH)], msg_v.at[b], sem_l
                ).wait()
                nxt = chunk + 16

                @pl.when(nxt < NCHUNK)
                def _():
                    pltpu.async_copy(
                        ridx_hbm.at[pl.ds(nxt * CH, CH)], idx_v.at[1 - b], sem_l
                    )
                    pltpu.async_copy(
                        m_hbm.at[cid, pl.ds(nxt * CH, CH)], msg_v.at[1 - b], sem_l
                    )

                pltpu.sync_copy(msg_v.at[b], acc.at[idx_v.at[b]], add=True)

        return carry

    lax.fori_loop(0, (NCHUNK // 16 + 2) // 2, spair, 0)
    plsc.subcore_barrier()

    def wbody(j, carry):
        pltpu.sync_copy(
            acc.at[pl.ds(row0 + j * CH, CH)], out_hbm.at[cid, pl.ds(row0 + j * CH, CH)]
        )
        return carry

    lax.fori_loop(0, Z_FULL, wbody, 0)
    pltpu.sync_copy(
        acc.at[pl.ds(row0 + Z_FULL * CH, Z_TAIL)],
        out_hbm.at[cid, pl.ds(row0 + Z_FULL * CH, Z_TAIL)],
    )


BE = 2000


def _edge_body(xj_ref, ef_ref, ea_ref, wn_ref, w1_ref, w2_ref, w3_ref, w4_ref,
               t4_ref, rr_ref, s_ref, wa_ref, wb_ref, wc_ref, wd_ref, m_ref):
    f32 = jnp.float32

    def dot(a, b):
        return jnp.dot(a, b, preferred_element_type=f32)

    def silu(x):
        return x * jax.nn.sigmoid(x)

    x = dot(xj_ref[...], wn_ref[...])
    sj = x[:, :C]
    vj = x[:, C:]
    h = silu(dot(ef_ref[...], w1_ref[...]))
    h = silu(dot(h, w2_ref[...]))
    h = silu(dot(h, w3_ref[...]))
    r = dot(h, w4_ref[...])
    wr1 = r[:, :C]
    wr2 = r[:, C:2 * C]
    wr3 = r[:, 2 * C:3 * C]
    wr4 = r[:, 3 * C:]
    ea = ea_ref[...]
    y0 = ea[:, 0:1]
    y1e = dot(ea, t4_ref[...])
    o0a = wr1 * sj * y0
    o0b = wr4 * dot(vj * y1e, s_ref[...])
    o1a = dot(wr2 * sj, rr_ref[...]) * y1e
    o1b = dot(wr3, rr_ref[...]) * vj * y0
    m = (dot(o0a, wa_ref[...]) + dot(o0b, wb_ref[...])
         + dot(o1a, wc_ref[...]) + dot(o1b, wd_ref[...]))
    m_ref[0] = m[:, :32]
    m_ref[1] = m[:, 32:]


def _edge_messages(xj, edge_feats, edge_attrs, weights):
    nblk = N_EDGES // BE
    w_specs = [
        pl.BlockSpec(w.shape, lambda i, nd=w.ndim: (0,) * nd) for w in weights
    ]
    return pl.pallas_call(
        _edge_body,
        grid=(nblk,),
        in_specs=[
            pl.BlockSpec((BE, 64), lambda i: (i, 0)),
            pl.BlockSpec((BE, 8), lambda i: (i, 0)),
            pl.BlockSpec((BE, 4), lambda i: (i, 0)),
        ] + w_specs,
        out_specs=pl.BlockSpec((2, BE, 32), lambda i: (0, i, 0)),
        out_shape=jax.ShapeDtypeStruct((2, N_EDGES, 32), jnp.float32),
        compiler_params=pltpu.CompilerParams(
            dimension_semantics=("arbitrary",),
        ),
    )(xj, edge_feats, edge_attrs, *weights)


def _prep_weights(W_node_0, W_node_1, mlp_W1, mlp_W2, mlp_W3, mlp_W4,
                  W_out_0, W_out_1):
    f32 = jnp.float32
    eye16 = jnp.eye(16, dtype=f32)
    wn = jnp.zeros((64, 64), f32)
    wn = wn.at[:16, :16].set(W_node_0 / 4.0)
    wn = wn.at[16:, 16:].set(jnp.kron(W_node_1, jnp.eye(3, dtype=f32)) / 4.0)
    w1 = mlp_W1 / jnp.sqrt(8.0)
    w2 = mlp_W2 / 8.0
    w3 = mlp_W3 / 8.0
    w4 = mlp_W4 / 8.0
    t4 = jnp.zeros((4, 16, 3), f32)
    for i in range(3):
        t4 = t4.at[1 + i, :, i].set(1.0)
    t4 = t4.reshape(4, 48)
    rr = jnp.kron(eye16, jnp.ones((1, 3), f32))
    sm = jnp.kron(eye16, jnp.ones((3, 1), f32)) / jnp.sqrt(3.0)
    scl = 1.0 / (jnp.sqrt(32.0) * 16.0)
    wa = jnp.zeros((16, 16, 4), f32).at[:, :, 0].set(W_out_0[:16] * scl).reshape(16, 64)
    wb = jnp.zeros((16, 16, 4), f32).at[:, :, 0].set(W_out_0[16:] * scl).reshape(16, 64)
    wc = jnp.zeros((16, 3, 16, 4), f32)
    wd = jnp.zeros((16, 3, 16, 4), f32)
    for i in range(3):
        wc = wc.at[:, i, :, 1 + i].set(W_out_1[:16] * scl)
        wd = wd.at[:, i, :, 1 + i].set(W_out_1[16:] * scl)
    wc = wc.reshape(48, 64)
    wd = wd.reshape(48, 64)
    return [wn, w1, w2, w3, w4, t4, rr, sm, wa, wb, wc, wd]


def kernel(node_feats, node_attrs, edge_attrs, edge_feats, edge_index,
           W_node_0, W_node_1, mlp_W1, mlp_W2, mlp_W3, mlp_W4, W_out_0, W_out_1):
    sender = edge_index[0]
    receiver = edge_index[1]
    weights = _prep_weights(W_node_0, W_node_1, mlp_W1, mlp_W2, mlp_W3, mlp_W4,
                            W_out_0, W_out_1)
    xj = _gather_kernel(node_feats, sender)
    m = _edge_messages(xj, edge_feats, edge_attrs, weights)
    zero = jnp.zeros((CH, 32), jnp.float32)
    out2 = _scatter_kernel(m, receiver, zero)
    return jnp.concatenate([out2[0], out2[1]], axis=1).reshape(N_NODES, C, 4)


# speedup vs baseline: 15.8832x; 15.8832x over previous
import functools

import jax
import jax.numpy as jnp
from jax import lax
from jax.experimental import pallas as pl
from jax.experimental.pallas import tpu as pltpu
from jax.experimental.pallas import tpu_sc as plsc

N_NODES = 50000
N_EDGES = 800000
C = 16

CH = 128
NW = 32
NS = 16
EPW = N_EDGES // NW
G_FULL = EPW // CH
G_TAIL = EPW - G_FULL * CH
NCHUNK = N_EDGES // CH
KMAX = (NCHUNK + NS - 1) // NS
RPT = 392
RPP = RPT * NS
NPADR = 2 * RPP
ACC_ROWS = RPP + 8
DUMP = RPP


def _gather_body(nodes_hbm, idx_hbm, out_hbm, idx_v, rows_v, tail_v, sem_g, sem_w):
    cid = lax.axis_index("c")
    sid = lax.axis_index("s")
    wid = sid * 2 + cid
    base = wid * EPW
    pltpu.sync_copy(idx_hbm.at[pl.ds(base, EPW)], idx_v)

    def pair(g, carry):
        for b in range(2):
            j = g * 2 + b

            @pl.when(j < G_FULL)
            def _(b=b, j=j):
                @pl.when(j >= 2)
                def _():
                    pltpu.make_async_copy(
                        nodes_hbm.at[pl.ds(0, CH)], rows_v.at[b], sem_w
                    ).wait()

                pltpu.async_copy(
                    nodes_hbm.at[idx_v.at[pl.ds(j * CH, CH)]], rows_v.at[b], sem_g
                ).wait()
                pltpu.async_copy(
                    rows_v.at[b], out_hbm.at[pl.ds(base + j * CH, CH)], sem_w
                )

        return carry

    lax.fori_loop(0, (G_FULL + 1) // 2, pair, 0)
    pltpu.make_async_copy(nodes_hbm.at[pl.ds(0, CH)], rows_v.at[0], sem_w).wait()
    pltpu.make_async_copy(nodes_hbm.at[pl.ds(0, CH)], rows_v.at[1], sem_w).wait()
    pltpu.async_copy(
        nodes_hbm.at[idx_v.at[pl.ds(G_FULL * CH, G_TAIL)]], tail_v, sem_g
    ).wait()
    pltpu.sync_copy(tail_v, out_hbm.at[pl.ds(base + G_FULL * CH, G_TAIL)])


def _scatter_body(m0_hbm, m1_hbm, ridx_hbm, zero_hbm, out0_hbm, out1_hbm,
                  idx_v, msg_v, zero_v, acc):
    cid = lax.axis_index("c")
    sid = lax.axis_index("s")
    row0 = sid * RPT
    pltpu.sync_copy(zero_hbm, zero_v)

    def scatter_pass(p, m_hbm):
        def chunk_body(k, carry):
            j = k * NS + sid

            @pl.when(j < NCHUNK)
            def _():
                pltpu.sync_copy(
                    ridx_hbm.at[pl.ds(p * N_EDGES + j * CH, CH)], idx_v
                )
                pltpu.sync_copy(m_hbm.at[pl.ds(j * CH, CH)], msg_v)
                pltpu.sync_copy(msg_v, acc.at[idx_v], add=True)

            return carry

        lax.fori_loop(0, KMAX, chunk_body, 0)

    def writeback(p, out_hbm):
        for j in range(3):
            pltpu.sync_copy(acc.at[pl.ds(row0 + j * CH, CH)], msg_v)
            pltpu.sync_copy(
                msg_v, out_hbm.at[pl.ds(p * RPP + row0 + j * CH, CH)]
            )
        pltpu.sync_copy(acc.at[pl.ds(row0 + 3 * CH, 8)], msg_v.at[pl.ds(0, 8)])
        pltpu.sync_copy(
            msg_v.at[pl.ds(0, 8)],
            out_hbm.at[pl.ds(p * RPP + row0 + 3 * CH, 8)],
        )

    for p in range(2):
        for j in range(3):
            pltpu.sync_copy(zero_v, acc.at[pl.ds(row0 + j * CH, CH)])
        pltpu.sync_copy(
            zero_v.at[pl.ds(0, 8)], acc.at[pl.ds(row0 + 3 * CH, 8)]
        )

        @pl.when(sid == 0)
        def _():
            pltpu.sync_copy(zero_v.at[pl.ds(0, 8)], acc.at[pl.ds(DUMP, 8)])

        plsc.subcore_barrier()

        @pl.when(cid == 0)
        def _(p=p):
            scatter_pass(p, m0_hbm)

        @pl.when(cid == 1)
        def _(p=p):
            scatter_pass(p, m1_hbm)

        plsc.subcore_barrier()

        @pl.when(cid == 0)
        def _(p=p):
            writeback(p, out0_hbm)

        @pl.when(cid == 1)
        def _(p=p):
            writeback(p, out1_hbm)

        plsc.subcore_barrier()


@functools.lru_cache(maxsize=None)
def _sc_kernels():
    mesh = plsc.VectorSubcoreMesh(core_axis_name="c", subcore_axis_name="s")
    gather = pl.kernel(
        _gather_body,
        mesh=mesh,
        out_type=jax.ShapeDtypeStruct((N_EDGES, 128), jnp.float32),
        scratch_types=[
            pltpu.VMEM((EPW,), jnp.int32),
            pltpu.VMEM((2, CH, 128), jnp.float32),
            pltpu.VMEM((G_TAIL, 128), jnp.float32),
            pltpu.SemaphoreType.DMA,
            pltpu.SemaphoreType.DMA,
        ],
    )
    scatter = pl.kernel(
        _scatter_body,
        mesh=mesh,
        out_type=(
            jax.ShapeDtypeStruct((NPADR, 128), jnp.float32),
            jax.ShapeDtypeStruct((NPADR, 128), jnp.float32),
        ),
        scratch_types=[
            pltpu.VMEM((CH,), jnp.int32),
            pltpu.VMEM((CH, 128), jnp.float32),
            pltpu.VMEM((CH, 128), jnp.float32),
            pltpu.VMEM_SHARED((ACC_ROWS, 128), jnp.float32),
        ],
    )
    return gather, scatter


BE = 2000


def _edge_body(xj_ref, ef_ref, ea_ref, oh_ref, wn_ref, w1_ref, w2_ref, w3_ref,
               w4_ref, t4_ref, rr_ref, s_ref, wa_ref, wb_ref, wc_ref, wd_ref,
               m0_ref, m1_ref):
    f32 = jnp.float32

    def dot(a, b):
        return jnp.dot(a, b, preferred_element_type=f32)

    def silu(x):
        return x * jax.nn.sigmoid(x)

    x = dot(xj_ref[...], wn_ref[...])
    sj = x[:, :C]
    vj = x[:, C:]
    h = silu(dot(ef_ref[...], w1_ref[...]))
    h = silu(dot(h, w2_ref[...]))
    h = silu(dot(h, w3_ref[...]))
    r = dot(h, w4_ref[...])
    wr1 = r[:, :C]
    wr2 = r[:, C:2 * C]
    wr3 = r[:, 2 * C:3 * C]
    wr4 = r[:, 3 * C:]
    ea = ea_ref[...]
    y0 = ea[:, 0:1]
    y1e = dot(ea, t4_ref[...])
    o0a = wr1 * sj * y0
    o0b = wr4 * dot(vj * y1e, s_ref[...])
    o1a = dot(wr2 * sj, rr_ref[...]) * y1e
    o1b = dot(wr3, rr_ref[...]) * vj * y0
    m = (dot(o0a, wa_ref[...]) + dot(o0b, wb_ref[...])
         + dot(o1a, wc_ref[...]) + dot(o1b, wd_ref[...]))
    oh = oh_ref[...]
    m0 = m[:, :32]
    m1 = m[:, 32:]
    m0_ref[...] = jnp.concatenate([m0 * oh[:, c:c + 1] for c in range(4)], axis=1)
    m1_ref[...] = jnp.concatenate([m1 * oh[:, c:c + 1] for c in range(4)], axis=1)


def _edge_messages(xj, edge_feats, edge_attrs, onehot, weights):
    nblk = N_EDGES // BE
    w_specs = [
        pl.BlockSpec(w.shape, lambda i, nd=w.ndim: (0,) * nd) for w in weights
    ]
    out_sds = jax.ShapeDtypeStruct((N_EDGES, 128), jnp.float32)
    return pl.pallas_call(
        _edge_body,
        grid=(nblk,),
        in_specs=[
            pl.BlockSpec((BE, 128), lambda i: (i, 0)),
            pl.BlockSpec((BE, 8), lambda i: (i, 0)),
            pl.BlockSpec((BE, 4), lambda i: (i, 0)),
            pl.BlockSpec((BE, 4), lambda i: (i, 0)),
        ] + w_specs,
        out_specs=[
            pl.BlockSpec((BE, 128), lambda i: (i, 0)),
            pl.BlockSpec((BE, 128), lambda i: (i, 0)),
        ],
        out_shape=[out_sds, out_sds],
        compiler_params=pltpu.CompilerParams(
            dimension_semantics=("arbitrary",),
        ),
    )(xj, edge_feats, edge_attrs, onehot, *weights)


def _prep_weights(W_node_0, W_node_1, mlp_W1, mlp_W2, mlp_W3, mlp_W4,
                  W_out_0, W_out_1):
    f32 = jnp.float32
    eye16 = jnp.eye(16, dtype=f32)
    wn = jnp.zeros((128, 64), f32)
    wn = wn.at[:16, :16].set(W_node_0 / 4.0)
    wn = wn.at[16:64, 16:].set(jnp.kron(W_node_1, jnp.eye(3, dtype=f32)) / 4.0)
    w1 = mlp_W1 / jnp.sqrt(8.0)
    w2 = mlp_W2 / 8.0
    w3 = mlp_W3 / 8.0
    w4 = mlp_W4 / 8.0
    t4 = jnp.zeros((4, 16, 3), f32)
    for i in range(3):
        t4 = t4.at[1 + i, :, i].set(1.0)
    t4 = t4.reshape(4, 48)
    rr = jnp.kron(eye16, jnp.ones((1, 3), f32))
    sm = jnp.kron(eye16, jnp.ones((3, 1), f32)) / jnp.sqrt(3.0)
    scl = 1.0 / (jnp.sqrt(32.0) * 16.0)
    wa = jnp.zeros((16, 16, 4), f32).at[:, :, 0].set(W_out_0[:16] * scl).reshape(16, 64)
    wb = jnp.zeros((16, 16, 4), f32).at[:, :, 0].set(W_out_0[16:] * scl).reshape(16, 64)
    wc = jnp.zeros((16, 3, 16, 4), f32)
    wd = jnp.zeros((16, 3, 16, 4), f32)
    for i in range(3):
        wc = wc.at[:, i, :, 1 + i].set(W_out_1[:16] * scl)
        wd = wd.at[:, i, :, 1 + i].set(W_out_1[16:] * scl)
    wc = wc.reshape(48, 64)
    wd = wd.reshape(48, 64)
    return [wn, w1, w2, w3, w4, t4, rr, sm, wa, wb, wc, wd]


def kernel(node_feats, node_attrs, edge_attrs, edge_feats, edge_index,
           W_node_0, W_node_1, mlp_W1, mlp_W2, mlp_W3, mlp_W4, W_out_0, W_out_1):
    sender = edge_index[0]
    receiver = edge_index[1]
    weights = _prep_weights(W_node_0, W_node_1, mlp_W1, mlp_W2, mlp_W3, mlp_W4,
                            W_out_0, W_out_1)
    gather_k, scatter_k = _sc_kernels()
    table = jnp.pad(node_feats, ((0, 0), (0, 64)))
    xj = gather_k(table, sender)
    rrow = (receiver // 4).astype(jnp.int32)
    onehot = (receiver[:, None] % 4 == jnp.arange(4)[None, :]).astype(jnp.float32)
    m0, m1 = _edge_messages(xj, edge_feats, edge_attrs, onehot, weights)
    ridx2 = jnp.concatenate([
        jnp.where(rrow < RPP, rrow, DUMP),
        jnp.where(rrow >= RPP, rrow - RPP, DUMP),
    ]).astype(jnp.int32)
    zero = jnp.zeros((CH, 128), jnp.float32)
    out0, out1 = scatter_k(m0, m1, ridx2, zero)
    o0 = out0.reshape(NPADR * 4, 32)[:N_NODES]
    o1 = out1.reshape(NPADR * 4, 32)[:N_NODES]
    return jnp.concatenate([o0, o1], axis=1).reshape(N_NODES, C, 4)
